# Initial kernel scaffold; baseline (speedup 1.0000x reference)
import functools

import jax
import jax.numpy as jnp
from jax import lax
from jax.experimental import pallas as pl
from jax.experimental.pallas import tpu as pltpu
from jax.experimental.pallas import tpu_sc as plsc

_N = 100000
_E = 3200000
_C = 16
_LAYERS = 3
_ALPHA = 0.9

_NC = 2
_NS = 16
_NW = _NC * _NS
_IDXW = 128
_JB = 8
_CHUNK = _IDXW * _JB
_CPW = -(-(_E // _NW) // _CHUNK)
_EPAD = _CPW * _CHUNK * _NW
_EROWS = _EPAD // _IDXW
_NACC = _N + 8
_NPT = _N // _NS
_ZROWS = 625

_mesh = plsc.VectorSubcoreMesh(
    core_axis_name="c", subcore_axis_name="s", num_cores=_NC, num_subcores=_NS
)


def _zero_my_acc_slice(zb, acc, sid):
    def zrow(i, carry):
        zb[i, :] = jnp.zeros((_C,), jnp.float32)
        return carry

    lax.fori_loop(0, _ZROWS, zrow, 0)
    base = sid * _NPT
    for k in range(_NPT // _ZROWS):
        pltpu.sync_copy(zb, acc.at[pl.ds(base + k * _ZROWS, _ZROWS)])


@functools.partial(
    pl.kernel,
    out_type=jax.ShapeDtypeStruct((_NC, _N, _C), jnp.float32),
    mesh=_mesh,
    scratch_types=[
        pltpu.VMEM((_JB, _IDXW), jnp.int32),
        pltpu.VMEM((_IDXW, _C), jnp.float32),
        pltpu.VMEM((_ZROWS, _C), jnp.float32),
        pltpu.VMEM_SHARED((_NACC, _C), jnp.float32),
        pltpu.SemaphoreType.DMA,
    ],
)
def _sc_degree(colp, out, colb, ones, zb, acc, ssem):
    c = lax.axis_index("c")
    sid = lax.axis_index("s")
    wid = c * _NS + sid

    def orow(i, carry):
        ones[i, :] = jnp.ones((_C,), jnp.float32)
        return carry

    lax.fori_loop(0, _IDXW, orow, 0)
    _zero_my_acc_slice(zb, acc, sid)
    plsc.subcore_barrier()

    rbase = wid * (_CPW * _JB)

    def chunk(t, carry):
        pltpu.sync_copy(colp.at[pl.ds(rbase + t * _JB, _JB)], colb)
        ds = [
            pltpu.async_copy(ones, acc.at[colb.at[j]], ssem, add=True)
            for j in range(_JB)
        ]
        for d in ds:
            d.wait()
        return carry

    lax.fori_loop(0, _CPW, chunk, 0)
    plsc.subcore_barrier()
    base = sid * _NPT
    pltpu.sync_copy(acc.at[pl.ds(base, _NPT)], out.at[c, pl.ds(base, _NPT)])


@functools.partial(
    pl.kernel,
    out_type=jax.ShapeDtypeStruct((_NC, _N, _C), jnp.float32),
    mesh=_mesh,
    scratch_types=[
        pltpu.VMEM((_JB, _IDXW), jnp.int32),
        pltpu.VMEM((_JB, _IDXW), jnp.int32),
        pltpu.VMEM((_JB, _IDXW, _C), jnp.float32),
        pltpu.VMEM((_ZROWS, _C), jnp.float32),
        pltpu.VMEM_SHARED((_NACC, _C), jnp.float32),
        pltpu.SemaphoreType.DMA,
        pltpu.SemaphoreType.DMA,
    ],
)
def _sc_propagate(rowp, colp, s, out, rowb, colb, msg, zb, acc, gsem, ssem):
    c = lax.axis_index("c")
    sid = lax.axis_index("s")
    wid = c * _NS + sid

    _zero_my_acc_slice(zb, acc, sid)
    plsc.subcore_barrier()

    rbase = wid * (_CPW * _JB)

    def chunk(t, carry):
        r0 = rbase + t * _JB
        pltpu.sync_copy(rowp.at[pl.ds(r0, _JB)], rowb)
        pltpu.sync_copy(colp.at[pl.ds(r0, _JB)], colb)
        gds = [
            pltpu.async_copy(s.at[rowb.at[j]], msg.at[j], gsem)
            for j in range(_JB)
        ]
        for d in gds:
            d.wait()
        sds = [
            pltpu.async_copy(msg.at[j], acc.at[colb.at[j]], ssem, add=True)
            for j in range(_JB)
        ]
        for d in sds:
            d.wait()
        return carry

    lax.fori_loop(0, _CPW, chunk, 0)
    plsc.subcore_barrier()
    base = sid * _NPT
    pltpu.sync_copy(acc.at[pl.ds(base, _NPT)], out.at[c, pl.ds(base, _NPT)])


_PB = 2500


def _prep_body(y, m, d0, d1, s0, res, dvb):
    deg = d0[...] + d1[...]
    dinv = jnp.where(deg > 0, lax.rsqrt(jnp.maximum(deg, 1.0)), 0.0)
    o = jnp.where(m[...] > 0, y[...], 0.0)
    res[...] = (1.0 - _ALPHA) * o
    s0[...] = dinv * o
    dvb[...] = dinv


_tc_prep = pl.pallas_call(
    _prep_body,
    grid=(_N // _PB,),
    in_specs=[
        pl.BlockSpec((_PB, _C), lambda i: (i, 0)),
        pl.BlockSpec((_PB, 1), lambda i: (i, 0)),
        pl.BlockSpec((_PB, _C), lambda i: (i, 0)),
        pl.BlockSpec((_PB, _C), lambda i: (i, 0)),
    ],
    out_specs=[
        pl.BlockSpec((_PB, _C), lambda i: (i, 0)),
        pl.BlockSpec((_PB, _C), lambda i: (i, 0)),
        pl.BlockSpec((_PB, _C), lambda i: (i, 0)),
    ],
    out_shape=[jax.ShapeDtypeStruct((_N, _C), jnp.float32)] * 3,
)

_UR = _N * _C // 128
_UB = 1250


def _upd_body(final, q0, q1, res, dvb, o):
    x = _ALPHA * dvb[...] * (q0[...] + q1[...]) + res[...]
    x = jnp.clip(x, 0.0, 1.0)
    o[...] = x if final else dvb[...] * x


def _make_update(final):
    return pl.pallas_call(
        functools.partial(_upd_body, final),
        grid=(_UR // _UB,),
        in_specs=[pl.BlockSpec((_UB, 128), lambda i: (i, 0))] * 4,
        out_specs=pl.BlockSpec((_UB, 128), lambda i: (i, 0)),
        out_shape=jax.ShapeDtypeStruct((_UR, 128), jnp.float32),
    )


_tc_update = _make_update(False)
_tc_update_final = _make_update(True)


def kernel(y, adj_t, train_mask):
    row = adj_t[0]
    col = adj_t[1]
    pad = _EPAD - _E
    rowp = jnp.concatenate([row, jnp.zeros((pad,), jnp.int32)])
    colp = jnp.concatenate([col, jnp.full((pad,), _N, jnp.int32)])
    rowp = rowp.reshape(_EROWS, _IDXW)
    colp = colp.reshape(_EROWS, _IDXW)

    degp = _sc_degree(colp)
    m = train_mask.astype(jnp.int32).reshape(_N, 1)
    s, res, dvb = _tc_prep(y, m, degp[0], degp[1])
    res128 = res.reshape(_UR, 128)
    dvb128 = dvb.reshape(_UR, 128)

    out = None
    for layer in range(_LAYERS):
        part = _sc_propagate(rowp, colp, s)
        q0 = part[0].reshape(_UR, 128)
        q1 = part[1].reshape(_UR, 128)
        if layer < _LAYERS - 1:
            s = _tc_update(q0, q1, res128, dvb128).reshape(_N, _C)
        else:
            out = _tc_update_final(q0, q1, res128, dvb128).reshape(_N, _C)
    return out


# speedup vs baseline: 15.2878x; 15.2878x over previous
import functools

import jax
import jax.numpy as jnp
from jax import lax
from jax.experimental import pallas as pl
from jax.experimental.pallas import tpu as pltpu
from jax.experimental.pallas import tpu_sc as plsc

_N = 100000
_E = 3200000
_C = 16
_LAYERS = 3
_ALPHA = 0.9

_NC = 2
_NS = 16
_IDXW = 128
_JB = 8
_CHUNK = _IDXW * _JB
_CPT = -(-_E // (_NS * _CHUNK))
_EPAD = _CPT * _CHUNK * _NS
_EROWS = _EPAD // _IDXW
_RPT = _EROWS // _NS
_NP = 100096
_HALF = _NP // 2
_DUMP = _HALF
_NACC = _HALF + 8
_APT = _HALF // _NS
_ZROWS = 391

_mesh = plsc.VectorSubcoreMesh(
    core_axis_name="c", subcore_axis_name="s", num_cores=_NC, num_subcores=_NS
)


def _zero_acc(zb, acc, sid):
    def zrow(i, carry):
        zb[i, :] = jnp.zeros((_C,), jnp.float32)
        return carry

    lax.fori_loop(0, _ZROWS, zrow, 0)
    base = sid * _APT
    for k in range(_APT // _ZROWS):
        pltpu.sync_copy(zb, acc.at[pl.ds(base + k * _ZROWS, _ZROWS)])

    @pl.when(sid == 0)
    def _():
        pltpu.sync_copy(zb.at[pl.ds(0, 8)], acc.at[pl.ds(_HALF, 8)])


def _load_cols(col0, col1, colb, c, r0):
    @pl.when(c == 0)
    def _():
        pltpu.sync_copy(col0.at[pl.ds(r0, _JB)], colb)

    @pl.when(c == 1)
    def _():
        pltpu.sync_copy(col1.at[pl.ds(r0, _JB)], colb)


def _dump_acc(acc, out, c, sid):
    base = sid * _APT
    dst = pl.multiple_of(c * _HALF + base, 8)
    pltpu.sync_copy(acc.at[pl.ds(base, _APT)], out.at[pl.ds(dst, _APT)])


@functools.partial(
    pl.kernel,
    out_type=jax.ShapeDtypeStruct((_NP, _C), jnp.float32),
    mesh=_mesh,
    scratch_types=[
        pltpu.VMEM((_JB, _IDXW), jnp.int32),
        pltpu.VMEM((_IDXW, _C), jnp.float32),
        pltpu.VMEM((_ZROWS, _C), jnp.float32),
        pltpu.VMEM_SHARED((_NACC, _C), jnp.float32),
        pltpu.SemaphoreType.DMA,
    ],
    compiler_params=pltpu.CompilerParams(use_tc_tiling_on_sc=False),
)
def _sc_degree(col0, col1, out, colb, ones, zb, acc, ssem):
    c = lax.axis_index("c")
    sid = lax.axis_index("s")

    def orow(i, carry):
        ones[i, :] = jnp.ones((_C,), jnp.float32)
        return carry

    lax.fori_loop(0, _IDXW, orow, 0)
    _zero_acc(zb, acc, sid)
    plsc.subcore_barrier()

    rbase = sid * _RPT

    def chunk(t, carry):
        _load_cols(col0, col1, colb, c, rbase + t * _JB)
        ds = [
            pltpu.async_copy(ones, acc.at[colb.at[j]], ssem, add=True)
            for j in range(_JB)
        ]
        for d in ds:
            d.wait()
        return carry

    lax.fori_loop(0, _CPT, chunk, 0)
    plsc.subcore_barrier()
    _dump_acc(acc, out, c, sid)


@functools.partial(
    pl.kernel,
    out_type=jax.ShapeDtypeStruct((_NP, _C), jnp.float32),
    mesh=_mesh,
    scratch_types=[
        pltpu.VMEM((_JB, _IDXW), jnp.int32),
        pltpu.VMEM((_JB, _IDXW), jnp.int32),
        pltpu.VMEM((_JB, _IDXW, _C), jnp.float32),
        pltpu.VMEM((_ZROWS, _C), jnp.float32),
        pltpu.VMEM_SHARED((_NACC, _C), jnp.float32),
        pltpu.SemaphoreType.DMA,
        pltpu.SemaphoreType.DMA,
    ],
    compiler_params=pltpu.CompilerParams(use_tc_tiling_on_sc=False),
)
def _sc_propagate(rowp, col0, col1, s, out, rowb, colb, msg, zb, acc, gsem, ssem):
    c = lax.axis_index("c")
    sid = lax.axis_index("s")

    _zero_acc(zb, acc, sid)
    plsc.subcore_barrier()

    rbase = sid * _RPT

    def chunk(t, carry):
        r0 = rbase + t * _JB
        pltpu.sync_copy(rowp.at[pl.ds(r0, _JB)], rowb)
        _load_cols(col0, col1, colb, c, r0)
        gds = [
            pltpu.async_copy(s.at[rowb.at[j]], msg.at[j], gsem)
            for j in range(_JB)
        ]
        for d in gds:
            d.wait()
        sds = [
            pltpu.async_copy(msg.at[j], acc.at[colb.at[j]], ssem, add=True)
            for j in range(_JB)
        ]
        for d in sds:
            d.wait()
        return carry

    lax.fori_loop(0, _CPT, chunk, 0)
    plsc.subcore_barrier()
    _dump_acc(acc, out, c, sid)


_IB = 1568


def _idx_body(colp, c0, c1):
    col = colp[...]
    c0[...] = jnp.where(col < _HALF, col, _DUMP)
    c1[...] = jnp.where(col >= _HALF, col - _HALF, _DUMP)


_tc_localize = pl.pallas_call(
    _idx_body,
    grid=(_EROWS // _IB,),
    in_specs=[pl.BlockSpec((_IB, _IDXW), lambda i: (i, 0))],
    out_specs=[
        pl.BlockSpec((_IB, _IDXW), lambda i: (i, 0)),
        pl.BlockSpec((_IB, _IDXW), lambda i: (i, 0)),
    ],
    out_shape=[jax.ShapeDtypeStruct((_EROWS, _IDXW), jnp.int32)] * 2,
)

_PB = 3128


def _prep_body(y, m, d, s0, res, dvb):
    deg = d[...]
    dinv = jnp.where(deg > 0, lax.rsqrt(jnp.maximum(deg, 1.0)), 0.0)
    o = jnp.where(m[...] > 0, y[...], 0.0)
    res[...] = (1.0 - _ALPHA) * o
    s0[...] = dinv * o
    dvb[...] = dinv


_tc_prep = pl.pallas_call(
    _prep_body,
    grid=(_NP // _PB,),
    in_specs=[
        pl.BlockSpec((_PB, _C), lambda i: (i, 0)),
        pl.BlockSpec((_PB, 1), lambda i: (i, 0)),
        pl.BlockSpec((_PB, _C), lambda i: (i, 0)),
    ],
    out_specs=[
        pl.BlockSpec((_PB, _C), lambda i: (i, 0)),
        pl.BlockSpec((_PB, _C), lambda i: (i, 0)),
        pl.BlockSpec((_PB, _C), lambda i: (i, 0)),
    ],
    out_shape=[jax.ShapeDtypeStruct((_NP, _C), jnp.float32)] * 3,
)


def _upd_body(final, q, res, dvb, o):
    x = _ALPHA * dvb[...] * q[...] + res[...]
    x = jnp.clip(x, 0.0, 1.0)
    o[...] = x if final else dvb[...] * x


def _make_update(final):
    return pl.pallas_call(
        functools.partial(_upd_body, final),
        grid=(_NP // _PB,),
        in_specs=[pl.BlockSpec((_PB, _C), lambda i: (i, 0))] * 3,
        out_specs=pl.BlockSpec((_PB, _C), lambda i: (i, 0)),
        out_shape=jax.ShapeDtypeStruct((_NP, _C), jnp.float32),
    )


_tc_update = _make_update(False)
_tc_update_final = _make_update(True)


def kernel(y, adj_t, train_mask):
    row = adj_t[0]
    col = adj_t[1]
    pad = _EPAD - _E
    rowp = jnp.concatenate([row, jnp.zeros((pad,), jnp.int32)])
    colp = jnp.concatenate([col, jnp.full((pad,), _N, jnp.int32)])
    rowp = rowp.reshape(_EROWS, _IDXW)
    colp = colp.reshape(_EROWS, _IDXW)
    col0, col1 = _tc_localize(colp)

    deg = _sc_degree(col0, col1)
    yp = jnp.concatenate([y, jnp.zeros((_NP - _N, _C), jnp.float32)])
    m = jnp.concatenate(
        [train_mask.astype(jnp.int32), jnp.zeros((_NP - _N,), jnp.int32)]
    ).reshape(_NP, 1)
    s, res, dvb = _tc_prep(yp, m, deg)

    out = None
    for layer in range(_LAYERS):
        q = _sc_propagate(rowp, col0, col1, s)
        if layer < _LAYERS - 1:
            s = _tc_update(q, res, dvb)
        else:
            out = _tc_update_final(q, res, dvb)
    return out[:_N]
